# baseline (speedup 1.0000x reference)
import math

import jax
import jax.numpy as jnp
import numpy as np
from jax.experimental import pallas as pl
from jax.experimental.pallas import tpu as pltpu

D, H, NB, L, K = 45, 256, 6, 2, 8
M = 3 * K - 1
DP = 48
HP = H + 1
MD = M * DP
TAIL = -math.log(1e-6)
MIN_BW = MIN_BH = MIN_D = 1e-3
LOG_Z = 0.5 * D * math.log(2.0 * math.pi)
T = 1024


def _np_masks():
    d_in = np.arange(1, D + 1)
    d_h = (np.arange(H) % (D - 1)) + 1
    m0 = (d_h[:, None] >= d_in[None, :]).astype(np.float32)
    mh = (d_h[:, None] >= d_h[None, :]).astype(np.float32)
    d_out = np.repeat(d_in, M)
    mf = (d_out[:, None] > d_h[None, :]).astype(np.float32)
    return m0, mh, mf


def _sel_chain(masks, vals):
    acc = vals[0]
    for mk, vk in zip(masks, vals[1:]):
        acc = jnp.where(mk, vk, acc)
    return acc


def _knots(u, mn):
    e = [jnp.exp(jax.lax.clamp(-60.0, uk, 60.0)) for uk in u]
    s = e[0]
    for ek in e[1:]:
        s = s + ek
    a = (2.0 * TAIL * (1.0 - mn * K)) / s
    cmn = 2.0 * TAIL * mn
    c = [jnp.full_like(u[0], -TAIL), e[0] * a + (cmn - TAIL)]
    wi = [e[k] * a + cmn for k in range(1, K - 1)]
    for k in range(1, K - 1):
        c.append(c[k] + wi[k - 1])
    c.append(jnp.full_like(u[0], TAIL))
    w = [c[1] + TAIL] + wi + [TAIL - c[K - 1]]
    return c, w


def _spline(z, p, valid_f, aug):
    xc = jnp.clip(z, -TAIL, TAIL)
    inside = z == xc

    uw = [p[m * DP:(m + 1) * DP, :] for m in range(K)]
    uh = [p[(K + m) * DP:(K + m + 1) * DP, :] for m in range(K)]
    ud = [p[(2 * K + j) * DP:(2 * K + j + 1) * DP, :] for j in range(K - 1)]

    cw, w = _knots(uw, MIN_BW)
    ch, h = _knots(uh, MIN_BH)
    d = [MIN_D + jax.nn.softplus(udj) for udj in ud]

    m = [xc >= cw[k] for k in range(1, K)]
    icw = _sel_chain(m, cw[:K])
    ibw = _sel_chain(m, w)
    ich = _sel_chain(m, ch[:K])
    ih = _sel_chain(m, h)
    id0 = _sel_chain(m, [jnp.ones_like(z)] + d)
    id1 = _sel_chain(m, d + [jnp.ones_like(z)])

    ribw = 1.0 / ibw
    idelta = ih * ribw
    th = (xc - icw) * ribw
    th1 = th * (1.0 - th)
    thsq = th * th
    den = idelta + (id0 + id1 - 2.0 * idelta) * th1
    rden = 1.0 / den
    y = ich + ih * (idelta * thsq + id0 * th1) * rden
    num = id1 * thsq + 2.0 * idelta * th1 + id0 * (1.0 - th) ** 2
    lad = jnp.log(idelta * idelta * num * rden * rden)

    y = jnp.where(inside, y, z)
    y = y * valid_f + aug
    lad = jnp.where(inside, lad, 0.0) * valid_f
    return y, lad


def _body(xt_ref, ct_ref, a0_ref, ah_ref, bh_ref, af_ref, bf_ref, o_ref):
    cnd = ct_ref[...]
    rows = jax.lax.broadcasted_iota(jnp.int32, (DP, 1), 0)
    valid_f = (rows < D).astype(jnp.float32)
    r45 = (rows == D).astype(jnp.float32)
    r46 = (rows == D + 1).astype(jnp.float32)
    aug = r45 + r46 * cnd
    z = xt_ref[...] + aug
    ld = jnp.zeros_like(z)
    for b in range(NB):
        hid = jnp.dot(a0_ref[b], z.astype(jnp.bfloat16),
                      preferred_element_type=jnp.float32
                      ).astype(jnp.bfloat16)
        for l in range(L):
            hid = jnp.maximum(
                jnp.dot(ah_ref[b, l], hid,
                        preferred_element_type=jnp.float32
                        ).astype(jnp.bfloat16) + bh_ref[b, l], 0.0)
        p = jnp.dot(af_ref[b], hid,
                    preferred_element_type=jnp.float32) + bf_ref[b]
        CH = T // 2
        ys, lads = [], []
        for c0 in range(0, T, CH):
            yc, ladc = _spline(z[:, c0:c0 + CH], p[:, c0:c0 + CH],
                               valid_f, aug[:, c0:c0 + CH])
            ys.append(yc)
            lads.append(ladc)
        z = jnp.concatenate(ys, axis=1)
        ld = ld + jnp.concatenate(lads, axis=1)
    out = jnp.sum(ld - 0.5 * z * z, axis=0, keepdims=True)
    o_ref[...] = out + (0.5 * (cnd * cnd + 1.0) - LOG_Z)


def kernel(x, cond, W0, b0, Wc, bc, Wh, bh, Wf, bf):
    B = x.shape[0]
    m0, mh, mf = _np_masks()

    a0 = jnp.concatenate(
        [W0 * m0, (b0 + bc)[..., None], Wc,
         jnp.zeros((NB, H, DP - D - 2), jnp.float32)], axis=-1)
    a0 = a0.astype(jnp.bfloat16)
    ah = (Wh * mh).astype(jnp.bfloat16)
    bh_r = bh[..., None].astype(jnp.bfloat16)
    psc = np.ones((M, 1, 1), np.float32)
    psc[:2 * K] = 1.0 / math.sqrt(H)
    af = (Wf * mf).reshape(NB, D, M, H).transpose(0, 2, 1, 3) * psc
    af = jnp.pad(af, ((0, 0), (0, 0), (0, DP - D), (0, 0)))
    af = af.reshape(NB, MD, H)
    af = af.astype(jnp.bfloat16)
    bf_r = bf.reshape(NB, D, M).transpose(0, 2, 1) * psc[..., 0]
    bf_r = jnp.pad(bf_r, ((0, 0), (0, 0), (0, DP - D)))
    bf_r = bf_r.reshape(NB, MD)[..., None]

    xt = jnp.pad(x, ((0, 0), (0, DP - D))).T
    ct = cond.T

    full = lambda shp: pl.BlockSpec(shp, lambda i: (0,) * len(shp))
    out = pl.pallas_call(
        _body,
        grid=(B // T,),
        in_specs=[
            pl.BlockSpec((DP, T), lambda i: (0, i)),
            pl.BlockSpec((1, T), lambda i: (0, i)),
            full((NB, H, DP)),
            full((NB, L, H, H)),
            full((NB, L, H, 1)),
            full((NB, MD, H)),
            full((NB, MD, 1)),
        ],
        out_specs=pl.BlockSpec((1, T), lambda i: (0, i)),
        out_shape=jax.ShapeDtypeStruct((1, B), jnp.float32),
        compiler_params=pltpu.CompilerParams(
            dimension_semantics=("parallel",),
        ),
    )(xt, ct, a0, ah, bh_r, af, bf_r)
    return out.reshape(B)


# speedup vs baseline: 1.1340x; 1.1340x over previous
import math

import jax
import jax.numpy as jnp
import numpy as np
from jax.experimental import pallas as pl
from jax.experimental.pallas import tpu as pltpu

D, H, NB, L, K = 45, 256, 6, 2, 8
M = 3 * K - 1
DP = 48
HP = H + 1
MD = M * DP
TAIL = -math.log(1e-6)
MIN_BW = MIN_BH = MIN_D = 1e-3
LOG_Z = 0.5 * D * math.log(2.0 * math.pi)
T = 1024


def _np_masks():
    d_in = np.arange(1, D + 1)
    d_h = (np.arange(H) % (D - 1)) + 1
    m0 = (d_h[:, None] >= d_in[None, :]).astype(np.float32)
    mh = (d_h[:, None] >= d_h[None, :]).astype(np.float32)
    d_out = np.repeat(d_in, M)
    mf = (d_out[:, None] > d_h[None, :]).astype(np.float32)
    return m0, mh, mf


def _sel_chain(masks, vals):
    acc = vals[0]
    for mk, vk in zip(masks, vals[1:]):
        acc = jnp.where(mk, vk, acc)
    return acc


def _knots(u, mn):
    e = [jnp.exp(jax.lax.clamp(-60.0, uk, 60.0)) for uk in u]
    s = e[0]
    for ek in e[1:]:
        s = s + ek
    a = (2.0 * TAIL * (1.0 - mn * K)) / s
    cmn = 2.0 * TAIL * mn
    c = [jnp.full_like(u[0], -TAIL), e[0] * a + (cmn - TAIL)]
    wi = [e[k] * a + cmn for k in range(1, K - 1)]
    for k in range(1, K - 1):
        c.append(c[k] + wi[k - 1])
    c.append(jnp.full_like(u[0], TAIL))
    w = [c[1] + TAIL] + wi + [TAIL - c[K - 1]]
    return c, w


def _spline(z, p, valid_f, aug):
    xc = jnp.clip(z, -TAIL, TAIL)
    inside = z == xc

    uw = [p[m * DP:(m + 1) * DP, :] for m in range(K)]
    uh = [p[(K + m) * DP:(K + m + 1) * DP, :] for m in range(K)]
    ud = [p[(2 * K + j) * DP:(2 * K + j + 1) * DP, :] for j in range(K - 1)]

    cw, w = _knots(uw, MIN_BW)
    ch, h = _knots(uh, MIN_BH)

    m = [xc >= cw[k] for k in range(1, K)]
    icw = _sel_chain(m, cw[:K])
    ibw = _sel_chain(m, w)
    ich = _sel_chain(m, ch[:K])
    ih = _sel_chain(m, h)
    cb = math.log(math.exp(1.0 - MIN_D) - 1.0)
    iud0 = _sel_chain(m, [jnp.full_like(z, cb)] + ud)
    iud1 = _sel_chain(m, ud + [jnp.full_like(z, cb)])
    id0 = MIN_D + jax.nn.softplus(iud0)
    id1 = MIN_D + jax.nn.softplus(iud1)

    ribw = 1.0 / ibw
    idelta = ih * ribw
    th = (xc - icw) * ribw
    th1 = th * (1.0 - th)
    thsq = th * th
    den = idelta + (id0 + id1 - 2.0 * idelta) * th1
    rden = 1.0 / den
    y = ich + ih * (idelta * thsq + id0 * th1) * rden
    num = id1 * thsq + 2.0 * idelta * th1 + id0 * (1.0 - th) ** 2
    lad = jnp.log(idelta * idelta * num * rden * rden)

    y = jnp.where(inside, y, z)
    y = y * valid_f + aug
    lad = jnp.where(inside, lad, 0.0) * valid_f
    return y, lad


def _body(xt_ref, ct_ref, a0_ref, ah_ref, bh_ref, af_ref, bf_ref, o_ref):
    cnd = ct_ref[...]
    rows = jax.lax.broadcasted_iota(jnp.int32, (DP, 1), 0)
    valid_f = (rows < D).astype(jnp.float32)
    r45 = (rows == D).astype(jnp.float32)
    r46 = (rows == D + 1).astype(jnp.float32)
    aug = r45 + r46 * cnd
    z = xt_ref[...] + aug
    ld = jnp.zeros_like(z)
    for b in range(NB):
        hid = jnp.dot(a0_ref[b], z.astype(jnp.bfloat16),
                      preferred_element_type=jnp.float32
                      ).astype(jnp.bfloat16)
        for l in range(L):
            hid = jnp.maximum(
                jnp.dot(ah_ref[b, l], hid,
                        preferred_element_type=jnp.float32
                        ).astype(jnp.bfloat16) + bh_ref[b, l], 0.0)
        p = jnp.dot(af_ref[b], hid,
                    preferred_element_type=jnp.float32) + bf_ref[b]
        z, lad = _spline(z, p, valid_f, aug)
        ld = ld + lad
    out = jnp.sum(ld - 0.5 * z * z, axis=0, keepdims=True)
    o_ref[...] = out + (0.5 * (cnd * cnd + 1.0) - LOG_Z)


def kernel(x, cond, W0, b0, Wc, bc, Wh, bh, Wf, bf):
    B = x.shape[0]
    m0, mh, mf = _np_masks()

    a0 = jnp.concatenate(
        [W0 * m0, (b0 + bc)[..., None], Wc,
         jnp.zeros((NB, H, DP - D - 2), jnp.float32)], axis=-1)
    a0 = a0.astype(jnp.bfloat16)
    ah = (Wh * mh).astype(jnp.bfloat16)
    bh_r = bh[..., None].astype(jnp.bfloat16)
    psc = np.ones((M, 1, 1), np.float32)
    psc[:2 * K] = 1.0 / math.sqrt(H)
    af = (Wf * mf).reshape(NB, D, M, H).transpose(0, 2, 1, 3) * psc
    af = jnp.pad(af, ((0, 0), (0, 0), (0, DP - D), (0, 0)))
    af = af.reshape(NB, MD, H)
    af = af.astype(jnp.bfloat16)
    bf_r = bf.reshape(NB, D, M).transpose(0, 2, 1) * psc[..., 0]
    bf_r = jnp.pad(bf_r, ((0, 0), (0, 0), (0, DP - D)))
    bf_r = bf_r.reshape(NB, MD)[..., None]

    xt = jnp.pad(x, ((0, 0), (0, DP - D))).T
    ct = cond.T

    full = lambda shp: pl.BlockSpec(shp, lambda i: (0,) * len(shp))
    out = pl.pallas_call(
        _body,
        grid=(B // T,),
        in_specs=[
            pl.BlockSpec((DP, T), lambda i: (0, i)),
            pl.BlockSpec((1, T), lambda i: (0, i)),
            full((NB, H, DP)),
            full((NB, L, H, H)),
            full((NB, L, H, 1)),
            full((NB, MD, H)),
            full((NB, MD, 1)),
        ],
        out_specs=pl.BlockSpec((1, T), lambda i: (0, i)),
        out_shape=jax.ShapeDtypeStruct((1, B), jnp.float32),
        compiler_params=pltpu.CompilerParams(
            dimension_semantics=("parallel",),
        ),
    )(xt, ct, a0, ah, bh_r, af, bf_r)
    return out.reshape(B)
